# Initial kernel scaffold; baseline (speedup 1.0000x reference)
import jax
import jax.numpy as jnp
from jax.experimental import pallas as pl


def kernel(x, params):
    raise NotImplementedError("write your pallas kernel here")


# speedup vs baseline: 2.2793x; 2.2793x over previous
import functools

import jax
import jax.numpy as jnp
from jax import lax
from jax.experimental import pallas as pl
from jax.experimental.pallas import tpu as pltpu
from jax.experimental.pallas import tpu_sc as plsc

B, L, D, DD, TARGET = 8, 2048, 1024, 256, 0.25
M = max(1, round(L * TARGET))
CHUNK = 256
NCHUNK = L // CHUNK

_HI = lax.Precision.HIGHEST


def _tc_body(tri_ref, x_ref, pos_ref, down_w_ref, down_b_ref, conv_w_ref,
             conv_b_ref, wr_w_ref, wr_b_ref, wi_w_ref, wi_b_ref, log_a_ref,
             out_w_ref, norm_w_ref, up_w_ref, up_b_ref, enc_ref, bp_ref,
             conv_buf, scan_carry):
    c = pl.program_id(1)

    tv = jax.lax.broadcasted_iota(jnp.int32, (M, CHUNK), 1) + c * CHUNK
    hit = (tv == pos_ref[...].reshape(M, 1)).astype(jnp.float32)
    bp_ref[0] = jnp.max(hit, axis=0, keepdims=True)

    @pl.when(c == 0)
    def _init():
        conv_buf[...] = jnp.zeros_like(conv_buf)
        scan_carry[...] = jnp.zeros_like(scan_carry)

    tri = tri_ref[...]
    h = lax.dot_general(x_ref[0], down_w_ref[...], (((1,), (1,)), ((), ())),
                        preferred_element_type=jnp.float32) + down_b_ref[...]
    for l in range(3):
        conv_buf[l, pl.ds(8, CHUNK), :] = h
        cw = conv_w_ref[l]
        xc = conv_b_ref[l] + h * cw[3:4]
        for k in range(3):
            xc = xc + conv_buf[l, pl.ds(5 + k, CHUNK), :] * cw[k:k + 1]
        conv_buf[l, pl.ds(5, 3), :] = conv_buf[l, pl.ds(CHUNK + 5, 3), :]

        r = jax.nn.sigmoid(
            lax.dot_general(xc, wr_w_ref[l], (((1,), (1,)), ((), ())),
                            preferred_element_type=jnp.float32) + wr_b_ref[l])
        i = jax.nn.sigmoid(
            lax.dot_general(xc, wi_w_ref[l], (((1,), (1,)), ((), ())),
                            preferred_element_type=jnp.float32) + wi_b_ref[l])
        a_base = jax.nn.sigmoid(log_a_ref[l])
        a_t = jnp.exp((8.0 * r) * jnp.log(a_base))
        loga = jnp.log(jnp.clip(a_t, 1e-07, None))
        raw = lax.dot_general(tri, loga, (((1,), (0,)), ((), ())),
                              preferred_element_type=jnp.float32,
                              precision=_HI) + scan_carry[l, 0:1, :]
        logA = jnp.clip(raw, -80.0, 0.0)
        b_t = jnp.sqrt(jnp.clip(1.0 - a_t * a_t, 1e-06, None)) * (i * xc)
        binv = b_t * jnp.exp(-logA)
        s = lax.dot_general(tri, binv, (((1,), (0,)), ((), ())),
                            preferred_element_type=jnp.float32,
                            precision=_HI) + scan_carry[l, 1:2, :]
        scan_carry[l, 0:1, :] = raw[CHUNK - 1:CHUNK, :]
        scan_carry[l, 1:2, :] = s[CHUNK - 1:CHUNK, :]
        y = jnp.exp(logA) * s
        out = lax.dot_general(y, out_w_ref[l], (((1,), (1,)), ((), ())),
                              preferred_element_type=jnp.float32)
        ms = jnp.mean(out * out, axis=1, keepdims=True)
        h = out * lax.rsqrt(ms + 1e-06) * norm_w_ref[l]
    enc_ref[0] = lax.dot_general(h, up_w_ref[...], (((1,), (1,)), ((), ())),
                                 preferred_element_type=jnp.float32) + up_b_ref[...]


def _encode_pallas(x, positions, pk):
    tri = jnp.tril(jnp.ones((CHUNK, CHUNK), jnp.float32))
    full = lambda s: pl.BlockSpec(s, lambda b, c: (0,) * len(s))
    in_specs = [
        full((CHUNK, CHUNK)),
        pl.BlockSpec((1, CHUNK, D), lambda b, c: (b, c, 0)),
        full((1, M)),
        full((DD, D)), full((1, DD)),
        full((3, 4, DD)), full((3, 1, DD)),
        full((3, DD, DD)), full((3, 1, DD)),
        full((3, DD, DD)), full((3, 1, DD)),
        full((3, 1, DD)),
        full((3, DD, DD)),
        full((3, 1, DD)),
        full((D, DD)), full((1, D)),
    ]
    return pl.pallas_call(
        _tc_body,
        grid=(B, NCHUNK),
        in_specs=in_specs,
        out_specs=[
            pl.BlockSpec((1, CHUNK, D), lambda b, c: (b, c, 0)),
            pl.BlockSpec((1, 1, CHUNK), lambda b, c: (b, 0, c)),
        ],
        out_shape=[
            jax.ShapeDtypeStruct((B, L, D), jnp.float32),
            jax.ShapeDtypeStruct((B, 1, L), jnp.float32),
        ],
        scratch_shapes=[
            pltpu.VMEM((3, CHUNK + 8, DD), jnp.float32),
            pltpu.VMEM((3, 8, DD), jnp.float32),
        ],
        compiler_params=pltpu.CompilerParams(
            dimension_semantics=("arbitrary", "arbitrary")),
    )(tri, x, positions[None, :], *pk)


def _stack_params(params):
    ls = params['layers']
    st = lambda name: jnp.stack([p[name] for p in ls])
    return (
        params['down_w'], params['down_b'][None, :],
        jnp.transpose(st('conv_w')[:, :, 0, :], (0, 2, 1)),
        st('conv_b')[:, None, :],
        st('wr_w'), st('wr_b')[:, None, :],
        st('wi_w'), st('wi_b')[:, None, :],
        st('log_a')[:, None, :],
        st('out_w'),
        st('norm_w')[:, None, :],
        params['up_w'], params['up_b'][None, :],
    )


_GC = 16


def _sc_route(enc_flat, flat_idx):
    info = plsc.get_sparse_core_info()
    nc, ns = info.num_cores, info.num_subcores
    nw = nc * ns
    rows_per_w = (B * M) // nw
    n_g = rows_per_w // _GC
    mesh = plsc.VectorSubcoreMesh(core_axis_name="c", subcore_axis_name="s")

    @functools.partial(
        pl.kernel, mesh=mesh,
        out_type=jax.ShapeDtypeStruct((B * M, D), jnp.float32),
        scratch_types=[
            pltpu.VMEM((rows_per_w,), jnp.int32),
            pltpu.VMEM((_GC, D), jnp.float32),
            pltpu.VMEM((_GC, D), jnp.float32),
            pltpu.SemaphoreType.DMA,
            pltpu.SemaphoreType.DMA,
        ],
    )
    def route(enc_hbm, idx_hbm, concept_hbm, idx_v, rows_a, rows_b, sem_a,
              sem_b):
        wid = lax.axis_index("s") * nc + lax.axis_index("c")
        base = wid * rows_per_w
        pltpu.sync_copy(idx_hbm.at[pl.ds(base, rows_per_w)], idx_v)
        bufs = (rows_a, rows_b)
        sems = (sem_a, sem_b)
        copies = [
            pltpu.make_async_copy(
                enc_hbm.at[idx_v.at[pl.ds(g * _GC, _GC)]],
                bufs[g % 2], sems[g % 2])
            for g in range(n_g)
        ]
        copies[0].start()
        for g in range(n_g):
            if g + 1 < n_g:
                copies[g + 1].start()
            copies[g].wait()
            pltpu.sync_copy(bufs[g % 2],
                            concept_hbm.at[pl.ds(base + g * _GC, _GC)])

    return route(enc_flat, flat_idx)


def kernel(x, params):
    pk = _stack_params(params)
    positions = jnp.linspace(0.0, L - 1, M).astype(jnp.int32)
    enc, bp3 = _encode_pallas(x, positions, pk)
    boundary_probs = bp3.reshape(B, L)

    flat_idx = (jnp.arange(B, dtype=jnp.int32)[:, None] * L
                + positions[None, :]).reshape(-1)
    concept_flat = _sc_route(enc.reshape(B * L, D), flat_idx)
    concept_tokens = concept_flat.reshape(B, M, D)
    boundary_idx = jnp.broadcast_to(positions[None, :], (B, M))
    concept_mask = jnp.ones((B, M), dtype=bool)
    return concept_tokens, enc, boundary_probs, boundary_idx, concept_mask


# speedup vs baseline: 4.0332x; 1.7694x over previous
import functools

import jax
import jax.numpy as jnp
from jax import lax
from jax.experimental import pallas as pl
from jax.experimental.pallas import tpu as pltpu
from jax.experimental.pallas import tpu_sc as plsc

B, L, D, DD, TARGET = 8, 2048, 1024, 256, 0.25
M = max(1, round(L * TARGET))
CHUNK = 128
NCHUNK = L // CHUNK
_LOG_EPS = -16.11809565095832

_HI = lax.Precision.HIGHEST


def _mm(a, w_t, prec=None):
    return lax.dot_general(a, w_t, (((1,), (1,)), ((), ())),
                           preferred_element_type=jnp.float32,
                           precision=prec)


def _tc_body(tri_ref, x_ref, pos_ref, down_w_ref, down_b_ref, conv_w_ref,
             conv_b_ref, wr_w_ref, wr_b_ref, wi_w_ref, wi_b_ref, log_a_ref,
             out_w_ref, norm_w_ref, up_w_ref, up_b_ref, enc_ref, bp_ref,
             conv_buf, scan_carry):
    c = pl.program_id(0)

    tv = jax.lax.broadcasted_iota(jnp.int32, (M, CHUNK), 1) + c * CHUNK
    hit = (tv == pos_ref[...].reshape(M, 1)).astype(jnp.float32)
    bp_ref[...] = jnp.max(hit, axis=0, keepdims=True)

    @pl.when(c == 0)
    def _init():
        conv_buf[...] = jnp.zeros_like(conv_buf)
        scan_carry[...] = jnp.zeros_like(scan_carry)

    tri = tri_ref[...]
    h = _mm(x_ref[...].reshape(B * CHUNK, D), down_w_ref[...]) + down_b_ref[...]
    for l in range(3):
        conv_buf[l, :, pl.ds(8, CHUNK), :] = h.reshape(B, CHUNK, DD)
        cw = conv_w_ref[l]
        xc3 = conv_b_ref[l][None] + h.reshape(B, CHUNK, DD) * cw[3:4][None]
        for k in range(3):
            xc3 = xc3 + conv_buf[l, :, pl.ds(5 + k, CHUNK), :] * cw[k:k + 1][None]
        conv_buf[l, :, pl.ds(5, 3), :] = conv_buf[l, :, pl.ds(CHUNK + 5, 3), :]
        xc = xc3.reshape(B * CHUNK, DD)

        r = jax.nn.sigmoid(_mm(xc, wr_w_ref[l]) + wr_b_ref[l])
        i = jax.nn.sigmoid(_mm(xc, wi_w_ref[l]) + wi_b_ref[l])
        a_base = jax.nn.sigmoid(log_a_ref[l])
        z = (8.0 * r) * jnp.log(a_base)
        a_t = jnp.exp(z)
        loga = jnp.maximum(z, _LOG_EPS).reshape(B, CHUNK, DD)
        raw = jnp.concatenate(
            [lax.dot_general(tri, loga[b], (((1,), (0,)), ((), ())),
                             preferred_element_type=jnp.float32,
                             precision=_HI)[None] for b in range(B)],
            axis=0) + scan_carry[l, :, 0:1, :]
        logA = jnp.clip(raw, -80.0, 0.0).reshape(B * CHUNK, DD)
        b_t = jnp.sqrt(jnp.clip(1.0 - a_t * a_t, 1e-06, None)) * (i * xc)
        binv = (b_t * jnp.exp(-logA)).reshape(B, CHUNK, DD)
        s = jnp.concatenate(
            [lax.dot_general(tri, binv[b], (((1,), (0,)), ((), ())),
                             preferred_element_type=jnp.float32,
                             precision=_HI)[None] for b in range(B)],
            axis=0) + scan_carry[l, :, 1:2, :]
        scan_carry[l, :, 0:1, :] = raw[:, CHUNK - 1:CHUNK, :]
        scan_carry[l, :, 1:2, :] = s[:, CHUNK - 1:CHUNK, :]
        y = jnp.exp(logA) * s.reshape(B * CHUNK, DD)
        out = _mm(y, out_w_ref[l])
        ms = jnp.mean(out * out, axis=1, keepdims=True)
        h = out * lax.rsqrt(ms + 1e-06) * norm_w_ref[l]
    enc = _mm(h, up_w_ref[...]) + up_b_ref[...]
    enc_ref[...] = enc.reshape(B, CHUNK, D)


def _encode_pallas(x, positions, pk):
    tri = jnp.tril(jnp.ones((CHUNK, CHUNK), jnp.float32))
    full = lambda s: pl.BlockSpec(s, lambda c: (0,) * len(s))
    in_specs = [
        full((CHUNK, CHUNK)),
        pl.BlockSpec((B, CHUNK, D), lambda c: (0, c, 0)),
        full((1, M)),
        full((DD, D)), full((1, DD)),
        full((3, 4, DD)), full((3, 1, DD)),
        full((3, DD, DD)), full((3, 1, DD)),
        full((3, DD, DD)), full((3, 1, DD)),
        full((3, 1, DD)),
        full((3, DD, DD)),
        full((3, 1, DD)),
        full((D, DD)), full((1, D)),
    ]
    return pl.pallas_call(
        _tc_body,
        grid=(NCHUNK,),
        in_specs=in_specs,
        out_specs=[
            pl.BlockSpec((B, CHUNK, D), lambda c: (0, c, 0)),
            pl.BlockSpec((1, CHUNK), lambda c: (0, c)),
        ],
        out_shape=[
            jax.ShapeDtypeStruct((B, L, D), jnp.float32),
            jax.ShapeDtypeStruct((1, L), jnp.float32),
        ],
        scratch_shapes=[
            pltpu.VMEM((3, B, CHUNK + 8, DD), jnp.float32),
            pltpu.VMEM((3, B, 8, DD), jnp.float32),
        ],
        compiler_params=pltpu.CompilerParams(
            dimension_semantics=("arbitrary",)),
    )(tri, x, positions[None, :], *pk)


def _stack_params(params):
    ls = params['layers']
    st = lambda name: jnp.stack([p[name] for p in ls])
    return (
        params['down_w'], params['down_b'][None, :],
        jnp.transpose(st('conv_w')[:, :, 0, :], (0, 2, 1)),
        st('conv_b')[:, None, :],
        st('wr_w'), st('wr_b')[:, None, :],
        st('wi_w'), st('wi_b')[:, None, :],
        st('log_a')[:, None, :],
        st('out_w'),
        st('norm_w')[:, None, :],
        params['up_w'], params['up_b'][None, :],
    )


_GC = 16


def _sc_route(enc_flat, flat_idx):
    info = plsc.get_sparse_core_info()
    nc, ns = info.num_cores, info.num_subcores
    nw = nc * ns
    rows_per_w = (B * M) // nw
    n_g = rows_per_w // _GC
    mesh = plsc.VectorSubcoreMesh(core_axis_name="c", subcore_axis_name="s")

    @functools.partial(
        pl.kernel, mesh=mesh,
        out_type=jax.ShapeDtypeStruct((B * M, D), jnp.float32),
        scratch_types=[
            pltpu.VMEM((rows_per_w,), jnp.int32),
            pltpu.VMEM((_GC, D), jnp.float32),
            pltpu.VMEM((_GC, D), jnp.float32),
            pltpu.SemaphoreType.DMA,
            pltpu.SemaphoreType.DMA,
        ],
    )
    def route(enc_hbm, idx_hbm, concept_hbm, idx_v, rows_a, rows_b, sem_a,
              sem_b):
        wid = lax.axis_index("s") * nc + lax.axis_index("c")
        base = wid * rows_per_w
        pltpu.sync_copy(idx_hbm.at[pl.ds(base, rows_per_w)], idx_v)
        bufs = (rows_a, rows_b)
        sems = (sem_a, sem_b)
        copies = [
            pltpu.make_async_copy(
                enc_hbm.at[idx_v.at[pl.ds(g * _GC, _GC)]],
                bufs[g % 2], sems[g % 2])
            for g in range(n_g)
        ]
        copies[0].start()
        for g in range(n_g):
            if g + 1 < n_g:
                copies[g + 1].start()
            copies[g].wait()
            pltpu.sync_copy(bufs[g % 2],
                            concept_hbm.at[pl.ds(base + g * _GC, _GC)])

    return route(enc_flat, flat_idx)


def kernel(x, params):
    pk = _stack_params(params)
    positions = jnp.linspace(0.0, L - 1, M).astype(jnp.int32)
    enc, bp_row = _encode_pallas(x, positions, pk)

    flat_idx = (jnp.arange(B, dtype=jnp.int32)[:, None] * L
                + positions[None, :]).reshape(-1)
    concept_flat = _sc_route(enc.reshape(B * L, D), flat_idx)
    concept_tokens = concept_flat.reshape(B, M, D)
    boundary_probs = jnp.broadcast_to(bp_row, (B, L))
    boundary_idx = jnp.broadcast_to(positions[None, :], (B, M))
    concept_mask = jnp.ones((B, M), dtype=bool)
    return concept_tokens, enc, boundary_probs, boundary_idx, concept_mask


# speedup vs baseline: 4.6298x; 1.1479x over previous
import functools

import jax
import jax.numpy as jnp
from jax import lax
from jax.experimental import pallas as pl
from jax.experimental.pallas import tpu as pltpu
from jax.experimental.pallas import tpu_sc as plsc

B, L, D, DD, TARGET = 8, 2048, 1024, 256, 0.25
M = max(1, round(L * TARGET))
CHUNK = 128
NCHUNK = L // CHUNK
_LOG_EPS = -16.11809565095832

_HI = lax.Precision.HIGHEST


def _mm(a, w_t, prec=None):
    return lax.dot_general(a, w_t, (((1,), (1,)), ((), ())),
                           preferred_element_type=jnp.float32,
                           precision=prec)


def _tc_body(tri_ref, x_ref, pos_ref, down_w_ref, down_b_ref, conv_w_ref,
             conv_b_ref, wr_w_ref, wr_b_ref, wi_w_ref, wi_b_ref, log_a_ref,
             out_w_ref, norm_w_ref, up_w_ref, up_b_ref, enc_ref, bp_ref,
             conv_buf, scan_carry):
    c = pl.program_id(0)

    tv = jax.lax.broadcasted_iota(jnp.int32, (M, CHUNK), 1) + c * CHUNK
    hit = (tv == pos_ref[...].reshape(M, 1)).astype(jnp.float32)
    bp_ref[...] = jnp.max(hit, axis=0, keepdims=True)

    @pl.when(c == 0)
    def _init():
        conv_buf[...] = jnp.zeros_like(conv_buf)
        scan_carry[...] = jnp.zeros_like(scan_carry)

    tri = tri_ref[...]
    h = _mm(x_ref[...].reshape(B * CHUNK, D), down_w_ref[...]) + down_b_ref[...]
    for l in range(3):
        conv_buf[l, :, pl.ds(8, CHUNK), :] = h.reshape(B, CHUNK, DD)
        cw = conv_w_ref[l]
        xc3 = conv_b_ref[l][None] + h.reshape(B, CHUNK, DD) * cw[3:4][None]
        for k in range(3):
            xc3 = xc3 + conv_buf[l, :, pl.ds(5 + k, CHUNK), :] * cw[k:k + 1][None]
        conv_buf[l, :, pl.ds(5, 3), :] = conv_buf[l, :, pl.ds(CHUNK + 5, 3), :]
        xc = xc3.reshape(B * CHUNK, DD)

        r = jax.nn.sigmoid(_mm(xc, wr_w_ref[l]) + wr_b_ref[l])
        i = jax.nn.sigmoid(_mm(xc, wi_w_ref[l]) + wi_b_ref[l])
        a_base = jax.nn.sigmoid(log_a_ref[l])
        z = (8.0 * r) * jnp.log(a_base)
        a_t = jnp.exp(z)
        loga = jnp.maximum(z, _LOG_EPS).reshape(B, CHUNK, DD)
        raw = jnp.concatenate(
            [lax.dot_general(tri, loga[b], (((1,), (0,)), ((), ())),
                             preferred_element_type=jnp.float32,
                             precision=_HI)[None] for b in range(B)],
            axis=0) + scan_carry[l, :, 0:1, :]
        logA = jnp.clip(raw, -80.0, 0.0).reshape(B * CHUNK, DD)
        amp = jnp.exp(logA)
        b_t = jnp.sqrt(jnp.clip(1.0 - a_t * a_t, 1e-06, None)) * (i * xc)
        binv = (b_t / amp).reshape(B, CHUNK, DD)
        s = jnp.concatenate(
            [lax.dot_general(tri, binv[b], (((1,), (0,)), ((), ())),
                             preferred_element_type=jnp.float32)[None]
             for b in range(B)],
            axis=0) + scan_carry[l, :, 1:2, :]
        scan_carry[l, :, 0:1, :] = raw[:, CHUNK - 1:CHUNK, :]
        scan_carry[l, :, 1:2, :] = s[:, CHUNK - 1:CHUNK, :]
        y = amp * s.reshape(B * CHUNK, DD)
        out = _mm(y, out_w_ref[l])
        ms = jnp.mean(out * out, axis=1, keepdims=True)
        h = out * lax.rsqrt(ms + 1e-06) * norm_w_ref[l]
    enc = _mm(h, up_w_ref[...]) + up_b_ref[...]
    enc_ref[...] = enc.reshape(B, CHUNK, D)


def _encode_pallas(x, positions, pk):
    tri = jnp.tril(jnp.ones((CHUNK, CHUNK), jnp.float32))
    full = lambda s: pl.BlockSpec(s, lambda c: (0,) * len(s))
    in_specs = [
        full((CHUNK, CHUNK)),
        pl.BlockSpec((B, CHUNK, D), lambda c: (0, c, 0)),
        full((1, M)),
        full((DD, D)), full((1, DD)),
        full((3, 4, DD)), full((3, 1, DD)),
        full((3, DD, DD)), full((3, 1, DD)),
        full((3, DD, DD)), full((3, 1, DD)),
        full((3, 1, DD)),
        full((3, DD, DD)),
        full((3, 1, DD)),
        full((D, DD)), full((1, D)),
    ]
    return pl.pallas_call(
        _tc_body,
        grid=(NCHUNK,),
        in_specs=in_specs,
        out_specs=[
            pl.BlockSpec((B, CHUNK, D), lambda c: (0, c, 0)),
            pl.BlockSpec((1, CHUNK), lambda c: (0, c)),
        ],
        out_shape=[
            jax.ShapeDtypeStruct((B, L, D), jnp.float32),
            jax.ShapeDtypeStruct((1, L), jnp.float32),
        ],
        scratch_shapes=[
            pltpu.VMEM((3, B, CHUNK + 8, DD), jnp.float32),
            pltpu.VMEM((3, B, 8, DD), jnp.float32),
        ],
        compiler_params=pltpu.CompilerParams(
            dimension_semantics=("arbitrary",)),
    )(tri, x, positions[None, :], *pk)


def _stack_params(params):
    ls = params['layers']
    st = lambda name: jnp.stack([p[name] for p in ls])
    return (
        params['down_w'], params['down_b'][None, :],
        jnp.transpose(st('conv_w')[:, :, 0, :], (0, 2, 1)),
        st('conv_b')[:, None, :],
        st('wr_w'), st('wr_b')[:, None, :],
        st('wi_w'), st('wi_b')[:, None, :],
        st('log_a')[:, None, :],
        st('out_w'),
        st('norm_w')[:, None, :],
        params['up_w'], params['up_b'][None, :],
    )


_GC = 16


def _sc_route(enc_flat, flat_idx):
    info = plsc.get_sparse_core_info()
    nc, ns = info.num_cores, info.num_subcores
    nw = nc * ns
    rows_per_w = (B * M) // nw
    n_g = rows_per_w // _GC
    mesh = plsc.VectorSubcoreMesh(core_axis_name="c", subcore_axis_name="s")

    @functools.partial(
        pl.kernel, mesh=mesh,
        out_type=jax.ShapeDtypeStruct((B * M, D), jnp.float32),
        scratch_types=[
            pltpu.VMEM((rows_per_w,), jnp.int32),
            pltpu.VMEM((_GC, D), jnp.float32),
            pltpu.VMEM((_GC, D), jnp.float32),
            pltpu.SemaphoreType.DMA,
            pltpu.SemaphoreType.DMA,
        ],
    )
    def route(enc_hbm, idx_hbm, concept_hbm, idx_v, rows_a, rows_b, sem_a,
              sem_b):
        wid = lax.axis_index("s") * nc + lax.axis_index("c")
        base = wid * rows_per_w
        pltpu.sync_copy(idx_hbm.at[pl.ds(base, rows_per_w)], idx_v)
        bufs = (rows_a, rows_b)
        sems = (sem_a, sem_b)
        copies = [
            pltpu.make_async_copy(
                enc_hbm.at[idx_v.at[pl.ds(g * _GC, _GC)]],
                bufs[g % 2], sems[g % 2])
            for g in range(n_g)
        ]
        copies[0].start()
        for g in range(n_g):
            if g + 1 < n_g:
                copies[g + 1].start()
            copies[g].wait()
            pltpu.sync_copy(bufs[g % 2],
                            concept_hbm.at[pl.ds(base + g * _GC, _GC)])

    return route(enc_flat, flat_idx)


def kernel(x, params):
    pk = _stack_params(params)
    positions = jnp.linspace(0.0, L - 1, M).astype(jnp.int32)
    enc, bp_row = _encode_pallas(x, positions, pk)

    flat_idx = (jnp.arange(B, dtype=jnp.int32)[:, None] * L
                + positions[None, :]).reshape(-1)
    concept_flat = _sc_route(enc.reshape(B * L, D), flat_idx)
    concept_tokens = concept_flat.reshape(B, M, D)
    boundary_probs = jnp.broadcast_to(bp_row, (B, L))
    boundary_idx = jnp.broadcast_to(positions[None, :], (B, M))
    concept_mask = jnp.ones((B, M), dtype=bool)
    return concept_tokens, enc, boundary_probs, boundary_idx, concept_mask


# speedup vs baseline: 4.7136x; 1.0181x over previous
import functools

import jax
import jax.numpy as jnp
from jax import lax
from jax.experimental import pallas as pl
from jax.experimental.pallas import tpu as pltpu
from jax.experimental.pallas import tpu_sc as plsc

B, L, D, DD, TARGET = 8, 2048, 1024, 256, 0.25
M = max(1, round(L * TARGET))
CHUNK = 128
NCHUNK = L // CHUNK
_LOG_EPS = -16.11809565095832


def _mm(a, w_t, prec=None):
    return lax.dot_general(a, w_t, (((1,), (1,)), ((), ())),
                           preferred_element_type=jnp.float32,
                           precision=prec)


def _bdot(tri_bf, v):
    return jnp.concatenate(
        [lax.dot_general(tri_bf, v[b], (((1,), (0,)), ((), ())),
                         preferred_element_type=jnp.float32)[None]
         for b in range(B)], axis=0)


def _csum(tri_bf, v):
    hi = v.astype(jnp.bfloat16)
    lo = (v - hi.astype(jnp.float32)).astype(jnp.bfloat16)
    return _bdot(tri_bf, hi) + _bdot(tri_bf, lo)


def _tc_body(tri_ref, x_ref, pos_ref, down_w_ref, down_b_ref, conv_w_ref,
             conv_b_ref, wr_w_ref, wr_b_ref, wi_w_ref, wi_b_ref, log_a_ref,
             out_w_ref, norm_w_ref, up_w_ref, up_b_ref, enc_ref, bp_ref,
             conv_buf, scan_carry):
    c = pl.program_id(0)

    tv = jax.lax.broadcasted_iota(jnp.int32, (M, CHUNK), 1) + c * CHUNK
    hit = (tv == pos_ref[...].reshape(M, 1)).astype(jnp.float32)
    bp_ref[...] = jnp.max(hit, axis=0, keepdims=True)

    @pl.when(c == 0)
    def _init():
        conv_buf[...] = jnp.zeros_like(conv_buf)
        scan_carry[...] = jnp.zeros_like(scan_carry)

    tri = tri_ref[...].astype(jnp.bfloat16)
    h = _mm(x_ref[...].reshape(B * CHUNK, D), down_w_ref[...]) + down_b_ref[...]
    for l in range(3):
        conv_buf[l, :, pl.ds(8, CHUNK), :] = h.reshape(B, CHUNK, DD)
        cw = conv_w_ref[l]
        xc3 = conv_b_ref[l][None] + h.reshape(B, CHUNK, DD) * cw[3:4][None]
        for k in range(3):
            xc3 = xc3 + conv_buf[l, :, pl.ds(5 + k, CHUNK), :] * cw[k:k + 1][None]
        conv_buf[l, :, pl.ds(5, 3), :] = conv_buf[l, :, pl.ds(CHUNK + 5, 3), :]
        xc = xc3.reshape(B * CHUNK, DD)

        r = jax.nn.sigmoid(_mm(xc, wr_w_ref[l]) + wr_b_ref[l])
        i = jax.nn.sigmoid(_mm(xc, wi_w_ref[l]) + wi_b_ref[l])
        a_base = jax.nn.sigmoid(log_a_ref[l])
        z = (8.0 * r) * jnp.log(a_base)
        a_t = jnp.exp(z)
        loga = jnp.maximum(z, _LOG_EPS).reshape(B, CHUNK, DD)
        raw = _csum(tri, loga) + scan_carry[l, :, 0:1, :]
        logA = jnp.clip(raw, -80.0, 0.0).reshape(B * CHUNK, DD)
        amp = jnp.exp(logA)
        b_t = jnp.sqrt(jnp.clip(1.0 - a_t * a_t, 1e-06, None)) * (i * xc)
        binv = (b_t / amp).reshape(B, CHUNK, DD)
        s = _csum(tri, binv) + scan_carry[l, :, 1:2, :]
        scan_carry[l, :, 0:1, :] = raw[:, CHUNK - 1:CHUNK, :]
        scan_carry[l, :, 1:2, :] = s[:, CHUNK - 1:CHUNK, :]
        y = amp * s.reshape(B * CHUNK, DD)
        out = _mm(y, out_w_ref[l])
        ms = jnp.mean(out * out, axis=1, keepdims=True)
        h = out * lax.rsqrt(ms + 1e-06) * norm_w_ref[l]
    enc = _mm(h, up_w_ref[...]) + up_b_ref[...]
    enc_ref[...] = enc.reshape(B, CHUNK, D)


def _encode_pallas(x, positions, pk):
    tri = jnp.tril(jnp.ones((CHUNK, CHUNK), jnp.float32))
    full = lambda s: pl.BlockSpec(s, lambda c: (0,) * len(s))
    in_specs = [
        full((CHUNK, CHUNK)),
        pl.BlockSpec((B, CHUNK, D), lambda c: (0, c, 0)),
        full((1, M)),
        full((DD, D)), full((1, DD)),
        full((3, 4, DD)), full((3, 1, DD)),
        full((3, DD, DD)), full((3, 1, DD)),
        full((3, DD, DD)), full((3, 1, DD)),
        full((3, 1, DD)),
        full((3, DD, DD)),
        full((3, 1, DD)),
        full((D, DD)), full((1, D)),
    ]
    return pl.pallas_call(
        _tc_body,
        grid=(NCHUNK,),
        in_specs=in_specs,
        out_specs=[
            pl.BlockSpec((B, CHUNK, D), lambda c: (0, c, 0)),
            pl.BlockSpec((1, CHUNK), lambda c: (0, c)),
        ],
        out_shape=[
            jax.ShapeDtypeStruct((B, L, D), jnp.float32),
            jax.ShapeDtypeStruct((1, L), jnp.float32),
        ],
        scratch_shapes=[
            pltpu.VMEM((3, B, CHUNK + 8, DD), jnp.float32),
            pltpu.VMEM((3, B, 8, DD), jnp.float32),
        ],
        compiler_params=pltpu.CompilerParams(
            dimension_semantics=("arbitrary",)),
    )(tri, x, positions[None, :], *pk)


def _stack_params(params):
    ls = params['layers']
    st = lambda name: jnp.stack([p[name] for p in ls])
    return (
        params['down_w'], params['down_b'][None, :],
        jnp.transpose(st('conv_w')[:, :, 0, :], (0, 2, 1)),
        st('conv_b')[:, None, :],
        st('wr_w'), st('wr_b')[:, None, :],
        st('wi_w'), st('wi_b')[:, None, :],
        st('log_a')[:, None, :],
        st('out_w'),
        st('norm_w')[:, None, :],
        params['up_w'], params['up_b'][None, :],
    )


_GC = 16


def _sc_route(enc_flat, flat_idx):
    info = plsc.get_sparse_core_info()
    nc, ns = info.num_cores, info.num_subcores
    nw = nc * ns
    rows_per_w = (B * M) // nw
    n_g = rows_per_w // _GC
    mesh = plsc.VectorSubcoreMesh(core_axis_name="c", subcore_axis_name="s")

    @functools.partial(
        pl.kernel, mesh=mesh,
        out_type=jax.ShapeDtypeStruct((B * M, D), jnp.float32),
        scratch_types=[
            pltpu.VMEM((rows_per_w,), jnp.int32),
            pltpu.VMEM((_GC, D), jnp.float32),
            pltpu.VMEM((_GC, D), jnp.float32),
            pltpu.SemaphoreType.DMA,
            pltpu.SemaphoreType.DMA,
        ],
    )
    def route(enc_hbm, idx_hbm, concept_hbm, idx_v, rows_a, rows_b, sem_a,
              sem_b):
        wid = lax.axis_index("s") * nc + lax.axis_index("c")
        base = wid * rows_per_w
        pltpu.sync_copy(idx_hbm.at[pl.ds(base, rows_per_w)], idx_v)
        bufs = (rows_a, rows_b)
        sems = (sem_a, sem_b)
        copies = [
            pltpu.make_async_copy(
                enc_hbm.at[idx_v.at[pl.ds(g * _GC, _GC)]],
                bufs[g % 2], sems[g % 2])
            for g in range(n_g)
        ]
        copies[0].start()
        for g in range(n_g):
            if g + 1 < n_g:
                copies[g + 1].start()
            copies[g].wait()
            pltpu.sync_copy(bufs[g % 2],
                            concept_hbm.at[pl.ds(base + g * _GC, _GC)])

    return route(enc_flat, flat_idx)


def kernel(x, params):
    pk = _stack_params(params)
    positions = jnp.linspace(0.0, L - 1, M).astype(jnp.int32)
    enc, bp_row = _encode_pallas(x, positions, pk)

    flat_idx = (jnp.arange(B, dtype=jnp.int32)[:, None] * L
                + positions[None, :]).reshape(-1)
    concept_flat = _sc_route(enc.reshape(B * L, D), flat_idx)
    concept_tokens = concept_flat.reshape(B, M, D)
    boundary_probs = jnp.broadcast_to(bp_row, (B, L))
    boundary_idx = jnp.broadcast_to(positions[None, :], (B, M))
    concept_mask = jnp.ones((B, M), dtype=bool)
    return concept_tokens, enc, boundary_probs, boundary_idx, concept_mask


# speedup vs baseline: 4.8413x; 1.0271x over previous
import functools

import jax
import jax.numpy as jnp
from jax import lax
from jax.experimental import pallas as pl
from jax.experimental.pallas import tpu as pltpu
from jax.experimental.pallas import tpu_sc as plsc

B, L, D, DD, TARGET = 8, 2048, 1024, 256, 0.25
M = max(1, round(L * TARGET))
CHUNK = 256
NCHUNK = L // CHUNK
_LOG_EPS = -16.11809565095832


def _mm(a, w_t, prec=None):
    return lax.dot_general(a, w_t, (((1,), (1,)), ((), ())),
                           preferred_element_type=jnp.float32,
                           precision=prec)


def _bdot(tri_bf, v):
    return jnp.concatenate(
        [lax.dot_general(tri_bf, v[b], (((1,), (0,)), ((), ())),
                         preferred_element_type=jnp.float32)[None]
         for b in range(B)], axis=0)


def _csum(tri_bf, v):
    hi = v.astype(jnp.bfloat16)
    lo = (v - hi.astype(jnp.float32)).astype(jnp.bfloat16)
    return _bdot(tri_bf, hi) + _bdot(tri_bf, lo)


def _tc_body(tri_ref, x_ref, pos_ref, down_w_ref, down_b_ref, conv_w_ref,
             conv_b_ref, wr_w_ref, wr_b_ref, wi_w_ref, wi_b_ref, log_a_ref,
             out_w_ref, norm_w_ref, up_w_ref, up_b_ref, enc_ref, bp_ref,
             conv_buf, scan_carry):
    c = pl.program_id(0)

    tv = jax.lax.broadcasted_iota(jnp.int32, (M, CHUNK), 1) + c * CHUNK
    hit = (tv == pos_ref[...].reshape(M, 1)).astype(jnp.float32)
    bp_ref[...] = jnp.max(hit, axis=0, keepdims=True)

    @pl.when(c == 0)
    def _init():
        conv_buf[...] = jnp.zeros_like(conv_buf)
        scan_carry[...] = jnp.zeros_like(scan_carry)

    tri = tri_ref[...].astype(jnp.bfloat16)
    h = _mm(x_ref[...].reshape(B * CHUNK, D), down_w_ref[...]) + down_b_ref[...]
    for l in range(3):
        conv_buf[l, :, pl.ds(8, CHUNK), :] = h.reshape(B, CHUNK, DD)
        cw = conv_w_ref[l]
        xc3 = conv_b_ref[l][None] + h.reshape(B, CHUNK, DD) * cw[3:4][None]
        for k in range(3):
            xc3 = xc3 + conv_buf[l, :, pl.ds(5 + k, CHUNK), :] * cw[k:k + 1][None]
        conv_buf[l, :, pl.ds(5, 3), :] = conv_buf[l, :, pl.ds(CHUNK + 5, 3), :]
        xc = xc3.reshape(B * CHUNK, DD)

        r = jax.nn.sigmoid(_mm(xc, wr_w_ref[l]) + wr_b_ref[l])
        i = jax.nn.sigmoid(_mm(xc, wi_w_ref[l]) + wi_b_ref[l])
        a_base = jax.nn.sigmoid(log_a_ref[l])
        z = (8.0 * r) * jnp.log(a_base)
        a_t = jnp.exp(z)
        loga = jnp.maximum(z, _LOG_EPS).reshape(B, CHUNK, DD)
        raw = _csum(tri, loga) + scan_carry[l, :, 0:1, :]
        logA = jnp.clip(raw, -80.0, 0.0).reshape(B * CHUNK, DD)
        amp = jnp.exp(logA)
        b_t = jnp.sqrt(jnp.clip(1.0 - a_t * a_t, 1e-06, None)) * (i * xc)
        binv = (b_t / amp).reshape(B, CHUNK, DD)
        s = _csum(tri, binv) + scan_carry[l, :, 1:2, :]
        scan_carry[l, :, 0:1, :] = raw[:, CHUNK - 1:CHUNK, :]
        scan_carry[l, :, 1:2, :] = s[:, CHUNK - 1:CHUNK, :]
        y = amp * s.reshape(B * CHUNK, DD)
        out = _mm(y, out_w_ref[l])
        ms = jnp.mean(out * out, axis=1, keepdims=True)
        h = out * lax.rsqrt(ms + 1e-06) * norm_w_ref[l]
    enc = _mm(h, up_w_ref[...]) + up_b_ref[...]
    enc_ref[...] = enc.reshape(B, CHUNK, D)


def _encode_pallas(x, positions, pk):
    tri = jnp.tril(jnp.ones((CHUNK, CHUNK), jnp.float32))
    full = lambda s: pl.BlockSpec(s, lambda c: (0,) * len(s))
    in_specs = [
        full((CHUNK, CHUNK)),
        pl.BlockSpec((B, CHUNK, D), lambda c: (0, c, 0)),
        full((1, M)),
        full((DD, D)), full((1, DD)),
        full((3, 4, DD)), full((3, 1, DD)),
        full((3, DD, DD)), full((3, 1, DD)),
        full((3, DD, DD)), full((3, 1, DD)),
        full((3, 1, DD)),
        full((3, DD, DD)),
        full((3, 1, DD)),
        full((D, DD)), full((1, D)),
    ]
    return pl.pallas_call(
        _tc_body,
        grid=(NCHUNK,),
        in_specs=in_specs,
        out_specs=[
            pl.BlockSpec((B, CHUNK, D), lambda c: (0, c, 0)),
            pl.BlockSpec((1, CHUNK), lambda c: (0, c)),
        ],
        out_shape=[
            jax.ShapeDtypeStruct((B, L, D), jnp.float32),
            jax.ShapeDtypeStruct((1, L), jnp.float32),
        ],
        scratch_shapes=[
            pltpu.VMEM((3, B, CHUNK + 8, DD), jnp.float32),
            pltpu.VMEM((3, B, 8, DD), jnp.float32),
        ],
        compiler_params=pltpu.CompilerParams(
            dimension_semantics=("arbitrary",)),
    )(tri, x, positions[None, :], *pk)


def _stack_params(params):
    ls = params['layers']
    st = lambda name: jnp.stack([p[name] for p in ls])
    return (
        params['down_w'], params['down_b'][None, :],
        jnp.transpose(st('conv_w')[:, :, 0, :], (0, 2, 1)),
        st('conv_b')[:, None, :],
        st('wr_w'), st('wr_b')[:, None, :],
        st('wi_w'), st('wi_b')[:, None, :],
        st('log_a')[:, None, :],
        st('out_w'),
        st('norm_w')[:, None, :],
        params['up_w'], params['up_b'][None, :],
    )


_GC = 16


def _sc_route(enc_flat, flat_idx):
    info = plsc.get_sparse_core_info()
    nc, ns = info.num_cores, info.num_subcores
    nw = nc * ns
    rows_per_w = (B * M) // nw
    n_g = rows_per_w // _GC
    mesh = plsc.VectorSubcoreMesh(core_axis_name="c", subcore_axis_name="s")

    @functools.partial(
        pl.kernel, mesh=mesh,
        out_type=jax.ShapeDtypeStruct((B * M, D), jnp.float32),
        scratch_types=[
            pltpu.VMEM((rows_per_w,), jnp.int32),
            pltpu.VMEM((_GC, D), jnp.float32),
            pltpu.VMEM((_GC, D), jnp.float32),
            pltpu.SemaphoreType.DMA,
            pltpu.SemaphoreType.DMA,
        ],
    )
    def route(enc_hbm, idx_hbm, concept_hbm, idx_v, rows_a, rows_b, sem_a,
              sem_b):
        wid = lax.axis_index("s") * nc + lax.axis_index("c")
        base = wid * rows_per_w
        pltpu.sync_copy(idx_hbm.at[pl.ds(base, rows_per_w)], idx_v)
        bufs = (rows_a, rows_b)
        sems = (sem_a, sem_b)
        copies = [
            pltpu.make_async_copy(
                enc_hbm.at[idx_v.at[pl.ds(g * _GC, _GC)]],
                bufs[g % 2], sems[g % 2])
            for g in range(n_g)
        ]
        copies[0].start()
        for g in range(n_g):
            if g + 1 < n_g:
                copies[g + 1].start()
            copies[g].wait()
            pltpu.sync_copy(bufs[g % 2],
                            concept_hbm.at[pl.ds(base + g * _GC, _GC)])

    return route(enc_flat, flat_idx)


def kernel(x, params):
    pk = _stack_params(params)
    positions = jnp.linspace(0.0, L - 1, M).astype(jnp.int32)
    enc, bp_row = _encode_pallas(x, positions, pk)

    flat_idx = (jnp.arange(B, dtype=jnp.int32)[:, None] * L
                + positions[None, :]).reshape(-1)
    concept_flat = _sc_route(enc.reshape(B * L, D), flat_idx)
    concept_tokens = concept_flat.reshape(B, M, D)
    boundary_probs = jnp.broadcast_to(bp_row, (B, L))
    boundary_idx = jnp.broadcast_to(positions[None, :], (B, M))
    concept_mask = jnp.ones((B, M), dtype=bool)
    return concept_tokens, enc, boundary_probs, boundary_idx, concept_mask
